# baseline (speedup 1.0000x reference)
import functools

import jax
import jax.numpy as jnp
from jax import lax
from jax.experimental import pallas as pl
from jax.experimental.pallas import tpu as pltpu
from jax.experimental.pallas import tpu_sc as plsc

_D = 768
_SEQ = 8192
_BATCH = 4

_NC = 2
_NS = 16
_NW = _NC * _NS
_SEQ_PER_W = _SEQ // _NW
_CHUNK_ROWS = 16
_N_CHUNKS = _SEQ_PER_W // _CHUNK_ROWS
_VECS_PER_ROW = _D // 16
_NBUF = 8
_N_GROUPS = _N_CHUNKS // 2


def _sc_body(we_hbm, pe_hbm, out_hbm, *scratch):
    pbufs, wbufs = scratch[0:2], scratch[2:2 + _NBUF]
    pe_sems = scratch[2 + _NBUF:4 + _NBUF]
    we_sems = scratch[4 + _NBUF:4 + 2 * _NBUF]
    out_sems = scratch[4 + 2 * _NBUF:4 + 3 * _NBUF]
    wid = lax.axis_index("s") * _NC + lax.axis_index("c")
    base_row = wid * _SEQ_PER_W

    def row0(c):
        return pl.multiple_of(base_row + c * _CHUNK_ROWS, 8)

    def we_desc(k, j):
        c, b = 2 * k + j // 4, j % 4
        return pltpu.make_async_copy(
            we_hbm.at[b, pl.ds(row0(c), _CHUNK_ROWS), :], wbufs[j], we_sems[j])

    def out_desc(k, j):
        c, b = 2 * k + j // 4, j % 4
        return pltpu.make_async_copy(
            wbufs[j], out_hbm.at[b, pl.ds(row0(c), _CHUNK_ROWS), :], out_sems[j])

    def pe_desc(c, parity):
        return pltpu.make_async_copy(
            pe_hbm.at[pl.ds(row0(c), _CHUNK_ROWS), :], pbufs[parity], pe_sems[parity])

    def group(k, first=False, last=False):
        for j in range(_NBUF):
            if j == 0:
                pe_desc(2 * k, 0).wait()
                pe_desc(2 * k + 1, 1).start()
            if j == 4:
                pe_desc(2 * k + 1, 1).wait()
                if not last:
                    pe_desc(2 * k + 2, 0).start()
            if j < 4:
                if not first:
                    out_desc(k - 1, j + 4).wait()
                we_desc(k, j + 4).start()
            elif not last:
                out_desc(k, j - 4).wait()
                we_desc(k + 1, j - 4).start()
            we_desc(k, j).wait()
            wbuf, pbuf = wbufs[j], pbufs[j // 4]

            @plsc.parallel_loop(0, _CHUNK_ROWS * _VECS_PER_ROW, 1, unroll=16)
            def _add(i):
                r = i // _VECS_PER_ROW
                v = (i - r * _VECS_PER_ROW) * 16
                plsc.addupdate(wbuf.at[r, pl.ds(v, 16)], pbuf[r, pl.ds(v, 16)])

            out_desc(k, j).start()

    pe_desc(0, 0).start()
    for j in range(4):
        we_desc(0, j).start()
    group(0, first=True)

    def body(k, carry):
        group(k)
        return carry

    lax.fori_loop(1, _N_GROUPS - 1, body, 0, unroll=False)
    group(_N_GROUPS - 1, last=True)
    for j in range(_NBUF):
        out_desc(_N_GROUPS - 1, j).wait()


_sc_add = functools.partial(
    pl.kernel,
    out_type=jax.ShapeDtypeStruct((_BATCH, _SEQ, _D), jnp.float32),
    mesh=plsc.VectorSubcoreMesh(core_axis_name="c", subcore_axis_name="s"),
    scratch_types=(
        [pltpu.VMEM((_CHUNK_ROWS, _D), jnp.float32)] * (2 + _NBUF)
        + [pltpu.SemaphoreType.DMA] * (2 + 2 * _NBUF)
    ),
)(_sc_body)


def kernel(word_embeddings, pe):
    return _sc_add(word_embeddings, pe)


# speedup vs baseline: 1.0184x; 1.0184x over previous
import functools

import jax
import jax.numpy as jnp
from jax import lax
from jax.experimental import pallas as pl
from jax.experimental.pallas import tpu as pltpu
from jax.experimental.pallas import tpu_sc as plsc

_D = 768
_SEQ = 8192
_BATCH = 4

_NC = 2
_NS = 16
_NW = _NC * _NS
_SEQ_PER_W = _SEQ // _NW
_CHUNK_ROWS = 16
_N_CHUNKS = _SEQ_PER_W // _CHUNK_ROWS
_VECS_PER_ROW = _D // 16
_NBUF = 8
_N_GROUPS = _N_CHUNKS // 2


def _sc_body(we_hbm, pe_hbm, out_hbm, *scratch):
    pbufs, wbufs = scratch[0:2], scratch[2:2 + _NBUF]
    pe_sems = scratch[2 + _NBUF:4 + _NBUF]
    we_sems = scratch[4 + _NBUF:4 + 2 * _NBUF]
    out_sems = scratch[4 + 2 * _NBUF:4 + 3 * _NBUF]
    wid = lax.axis_index("s") * _NC + lax.axis_index("c")
    base_row = wid * _SEQ_PER_W

    def row0(c):
        return pl.multiple_of(base_row + c * _CHUNK_ROWS, 8)

    def we_desc(k, j):
        c, b = 2 * k + j // 4, j % 4
        return pltpu.make_async_copy(
            we_hbm.at[b, pl.ds(row0(c), _CHUNK_ROWS), :], wbufs[j], we_sems[j])

    def out_desc(k, j):
        c, b = 2 * k + j // 4, j % 4
        return pltpu.make_async_copy(
            wbufs[j], out_hbm.at[b, pl.ds(row0(c), _CHUNK_ROWS), :], out_sems[j])

    def pe_desc(c, parity):
        return pltpu.make_async_copy(
            pe_hbm.at[pl.ds(row0(c), _CHUNK_ROWS), :], pbufs[parity], pe_sems[parity])

    def group(k, first=False, last=False):
        for j in range(_NBUF):
            if j == 0:
                pe_desc(2 * k, 0).wait()
                pe_desc(2 * k + 1, 1).start()
            if j == 4:
                pe_desc(2 * k + 1, 1).wait()
                if not last:
                    pe_desc(2 * k + 2, 0).start()
            if j < 4:
                if not first:
                    out_desc(k - 1, j + 4).wait()
                we_desc(k, j + 4).start()
            elif not last:
                out_desc(k, j - 4).wait()
                we_desc(k + 1, j - 4).start()
            we_desc(k, j).wait()
            wbuf, pbuf = wbufs[j], pbufs[j // 4]

            @plsc.parallel_loop(0, _CHUNK_ROWS * _VECS_PER_ROW, 1, unroll=8)
            def _add(i):
                r = i // _VECS_PER_ROW
                v = (i - r * _VECS_PER_ROW) * 16
                plsc.addupdate(wbuf.at[r, pl.ds(v, 16)], pbuf[r, pl.ds(v, 16)])

            out_desc(k, j).start()

    pe_desc(0, 0).start()
    for j in range(4):
        we_desc(0, j).start()
    group(0, first=True)

    def body(k, carry):
        group(k)
        return carry

    lax.fori_loop(1, _N_GROUPS - 1, body, 0, unroll=False)
    group(_N_GROUPS - 1, last=True)
    for j in range(_NBUF):
        out_desc(_N_GROUPS - 1, j).wait()


_sc_add = functools.partial(
    pl.kernel,
    out_type=jax.ShapeDtypeStruct((_BATCH, _SEQ, _D), jnp.float32),
    mesh=plsc.VectorSubcoreMesh(core_axis_name="c", subcore_axis_name="s"),
    scratch_types=(
        [pltpu.VMEM((_CHUNK_ROWS, _D), jnp.float32)] * (2 + _NBUF)
        + [pltpu.SemaphoreType.DMA] * (2 + 2 * _NBUF)
    ),
)(_sc_body)


def kernel(word_embeddings, pe):
    return _sc_add(word_embeddings, pe)
